# baseline (speedup 1.0000x reference)
import functools

import jax
import jax.numpy as jnp
from jax import lax
from jax.experimental import pallas as pl
from jax.experimental.pallas import tpu as pltpu
from jax.experimental.pallas import tpu_sc as plsc

_INFO = plsc.get_sparse_core_info()
_NC = _INFO.num_cores
_NS = _INFO.num_subcores
_NW = _NC * _NS

_TPAD = 24
_EPC = 1
_NBUF = 4


def _gather_rows(nb: int, t: int, d_pad: int):
    rows_per_w = nb * _TPAD // _NW
    chunk = _EPC * _TPAD
    n_chunks = rows_per_w // chunk
    n_rounds = n_chunks // _NBUF
    mesh = plsc.VectorSubcoreMesh(core_axis_name="c", subcore_axis_name="s")

    @functools.partial(
        pl.kernel,
        mesh=mesh,
        out_type=jax.ShapeDtypeStruct((nb * _TPAD, d_pad), jnp.float32),
        scratch_types=[
            pltpu.VMEM((rows_per_w,), jnp.int32),
            pltpu.VMEM((_NBUF, chunk, d_pad), jnp.float32),
        ]
        + [pltpu.SemaphoreType.DMA] * (2 * _NBUF),
    )
    def k(idx_hbm, table_hbm, out_hbm, idx_v, rows_v, *sems):
        gs = sems[:_NBUF]
        os = sems[_NBUF:]
        wid = lax.axis_index("s") * _NC + lax.axis_index("c")
        base_r = wid * rows_per_w
        pltpu.sync_copy(idx_hbm.at[pl.ds(base_r, rows_per_w)], idx_v)

        def gather(i, buf):
            pltpu.async_copy(
                table_hbm.at[idx_v.at[pl.ds(i * chunk, chunk)]],
                rows_v.at[buf],
                gs[buf],
            )

        def gather_wait(buf):
            pltpu.make_async_copy(
                table_hbm.at[idx_v.at[pl.ds(0, chunk)]], rows_v.at[buf], gs[buf]
            ).wait()

        def writeout(i, buf):
            pltpu.async_copy(
                rows_v.at[buf], out_hbm.at[pl.ds(base_r + i * chunk, chunk)],
                os[buf],
            )

        def write_wait(buf):
            pltpu.make_async_copy(
                rows_v.at[buf], out_hbm.at[pl.ds(base_r, chunk)], os[buf]
            ).wait()

        for b in range(_NBUF):
            gather(b, b)

        def body(j, carry):
            i0 = j * _NBUF
            for b in range(_NBUF):
                gather_wait(b)
                writeout(i0 + b, b)
            for b in range(_NBUF):
                write_wait(b)
                gather(i0 + _NBUF + b, b)
            return carry

        lax.fori_loop(0, n_rounds - 1, body, 0)

        i0 = (n_rounds - 1) * _NBUF
        for b in range(_NBUF):
            gather_wait(b)
            writeout(i0 + b, b)
        for b in range(_NBUF):
            write_wait(b)

    return k


def kernel(x, embedding_table):
    b, t = x.shape
    v, d = embedding_table.shape
    d_pad = (d + 127) // 128 * 128
    idx = jnp.pad(x.astype(jnp.int32), ((0, 0), (0, _TPAD - t))).reshape(-1)
    table_p = jnp.pad(embedding_table, ((0, 0), (0, d_pad - d)))
    out = _gather_rows(b, t, d_pad)(idx, table_p)
    return out.reshape(b, _TPAD, d_pad)[:, :t, :d]


# speedup vs baseline: 2.5371x; 2.5371x over previous
import functools

import jax
import jax.numpy as jnp
from jax import lax
from jax.experimental import pallas as pl
from jax.experimental.pallas import tpu as pltpu
from jax.experimental.pallas import tpu_sc as plsc

_INFO = plsc.get_sparse_core_info()
_NC = _INFO.num_cores
_NS = _INFO.num_subcores
_NW = _NC * _NS

_TPAD = 24
_EPC = 1
_NBUF = 4


def _gather_rows(nb: int, t: int, d_pad: int):
    rows_per_w = nb * _TPAD // _NW
    chunk = _EPC * _TPAD
    n_chunks = rows_per_w // chunk
    n_rounds = n_chunks // _NBUF
    mesh = plsc.VectorSubcoreMesh(core_axis_name="c", subcore_axis_name="s")

    @functools.partial(
        pl.kernel,
        mesh=mesh,
        out_type=jax.ShapeDtypeStruct((nb * _TPAD, d_pad), jnp.float32),
        scratch_types=[
            pltpu.VMEM((rows_per_w,), jnp.int32),
            pltpu.VMEM((_NBUF, chunk, d_pad), jnp.float32),
        ]
        + [pltpu.SemaphoreType.DMA] * (2 * _NBUF),
    )
    def k(idx_hbm, table_hbm, out_hbm, idx_v, rows_v, *sems):
        gs = sems[:_NBUF]
        os = sems[_NBUF:]
        wid = lax.axis_index("s") * _NC + lax.axis_index("c")
        base_r = wid * rows_per_w
        pltpu.sync_copy(idx_hbm.at[pl.ds(base_r, rows_per_w)], idx_v)

        def gather(i, buf):
            pltpu.async_copy(
                table_hbm.at[idx_v.at[pl.ds(i * chunk, chunk)]],
                rows_v.at[buf],
                gs[buf],
            )

        def gather_wait(buf):
            pltpu.make_async_copy(
                table_hbm.at[idx_v.at[pl.ds(0, chunk)]], rows_v.at[buf], gs[buf]
            ).wait()

        def writeout(i, buf):
            pltpu.async_copy(
                rows_v.at[buf], out_hbm.at[pl.ds(base_r + i * chunk, chunk)],
                os[buf],
            )

        def write_wait(buf):
            pltpu.make_async_copy(
                rows_v.at[buf], out_hbm.at[pl.ds(base_r, chunk)], os[buf]
            ).wait()

        for b in range(_NBUF):
            gather(b, b)

        def body(j, carry):
            i0 = j * _NBUF
            for b in range(_NBUF):
                gather_wait(b)
                writeout(i0 + b, b)
            for b in range(_NBUF):
                write_wait(b)
                gather(i0 + _NBUF + b, b)
            return carry

        lax.fori_loop(0, n_rounds - 1, body, 0)

        i0 = (n_rounds - 1) * _NBUF
        for b in range(_NBUF):
            gather_wait(b)
            writeout(i0 + b, b)
        for b in range(_NBUF):
            write_wait(b)

    return k


def kernel(x, embedding_table):
    b, t = x.shape
    v, d = embedding_table.shape
    d_pad = (d + 127) // 128 * 128
    xi = x.astype(jnp.int32)
    idx = jnp.concatenate([xi, xi[:, : _TPAD - t]], axis=1).reshape(-1)
    table_p = jnp.pad(embedding_table, ((0, 0), (0, d_pad - d)))
    out = _gather_rows(b, t, d_pad)(idx, table_p)
    return out.reshape(b, _TPAD, d_pad)[:, :t, :d]
